# baseline (speedup 1.0000x reference)
import jax
import jax.numpy as jnp
from jax.experimental import pallas as pl
from jax.experimental.pallas import tpu as pltpu

BB = 4
NBUF = 4


def _body(patch_hbm, pos_hbm, out_hbm, pos_v, ibufs, obufs, sem_pos,
          sem_in, sem_out):
    nchunk = patch_hbm.shape[0] // BB

    def in_copy(i):
        slot = i % NBUF
        return pltpu.make_async_copy(
            patch_hbm.at[pl.ds(i * BB, BB)], ibufs.at[slot], sem_in.at[slot]
        )

    def out_copy(i):
        slot = i % NBUF
        return pltpu.make_async_copy(
            obufs.at[slot], out_hbm.at[pl.ds(i * BB, BB)], sem_out.at[slot]
        )

    pos_copy = pltpu.make_async_copy(pos_hbm, pos_v, sem_pos)
    pos_copy.start()
    for i in range(NBUF):
        in_copy(i).start()
    pos_copy.wait()

    for i in range(nchunk):
        slot = i % NBUF
        in_copy(i).wait()
        if i >= NBUF:
            out_copy(i - NBUF).wait()
        obufs[slot] = ibufs[slot] + pos_v[...]
        out_copy(i).start()
        if i + NBUF < nchunk:
            in_copy(i + NBUF).start()

    for i in range(max(0, nchunk - NBUF), nchunk):
        out_copy(i).wait()


def kernel(patch, position_embedding):
    B, P, D = patch.shape
    pt = jnp.transpose(patch, (0, 2, 1))
    post = jnp.transpose(position_embedding, (1, 0))
    out = pl.pallas_call(
        _body,
        in_specs=[
            pl.BlockSpec(memory_space=pl.ANY),
            pl.BlockSpec(memory_space=pl.ANY),
        ],
        out_specs=pl.BlockSpec(memory_space=pl.ANY),
        out_shape=jax.ShapeDtypeStruct((B, D, P), patch.dtype),
        scratch_shapes=[
            pltpu.VMEM((D, P), patch.dtype),
            pltpu.VMEM((NBUF, BB, D, P), patch.dtype),
            pltpu.VMEM((NBUF, BB, D, P), patch.dtype),
            pltpu.SemaphoreType.DMA,
            pltpu.SemaphoreType.DMA((NBUF,)),
            pltpu.SemaphoreType.DMA((NBUF,)),
        ],
    )(pt, post)
    return jnp.transpose(out, (0, 2, 1))


# speedup vs baseline: 1.0381x; 1.0381x over previous
import jax
import jax.numpy as jnp
from jax.experimental import pallas as pl
from jax.experimental.pallas import tpu as pltpu

BB = 8
NBUF = 4


def _body(patch_hbm, pos_hbm, out_hbm, pos_v, ibufs, obufs, sem_pos,
          sem_in, sem_out):
    nchunk = patch_hbm.shape[0] // BB

    def in_copy(i):
        slot = i % NBUF
        return pltpu.make_async_copy(
            patch_hbm.at[pl.ds(i * BB, BB)], ibufs.at[slot], sem_in.at[slot]
        )

    def out_copy(i):
        slot = i % NBUF
        return pltpu.make_async_copy(
            obufs.at[slot], out_hbm.at[pl.ds(i * BB, BB)], sem_out.at[slot]
        )

    pos_copy = pltpu.make_async_copy(pos_hbm, pos_v, sem_pos)
    pos_copy.start()
    for i in range(NBUF):
        in_copy(i).start()
    pos_copy.wait()

    for i in range(nchunk):
        slot = i % NBUF
        in_copy(i).wait()
        if i >= NBUF:
            out_copy(i - NBUF).wait()
        obufs[slot] = ibufs[slot] + pos_v[...]
        out_copy(i).start()
        if i + NBUF < nchunk:
            in_copy(i + NBUF).start()

    for i in range(max(0, nchunk - NBUF), nchunk):
        out_copy(i).wait()


def kernel(patch, position_embedding):
    B, P, D = patch.shape
    pt = jnp.transpose(patch, (0, 2, 1))
    post = jnp.transpose(position_embedding, (1, 0))
    out = pl.pallas_call(
        _body,
        in_specs=[
            pl.BlockSpec(memory_space=pl.ANY),
            pl.BlockSpec(memory_space=pl.ANY),
        ],
        out_specs=pl.BlockSpec(memory_space=pl.ANY),
        out_shape=jax.ShapeDtypeStruct((B, D, P), patch.dtype),
        scratch_shapes=[
            pltpu.VMEM((D, P), patch.dtype),
            pltpu.VMEM((NBUF, BB, D, P), patch.dtype),
            pltpu.VMEM((NBUF, BB, D, P), patch.dtype),
            pltpu.SemaphoreType.DMA,
            pltpu.SemaphoreType.DMA((NBUF,)),
            pltpu.SemaphoreType.DMA((NBUF,)),
        ],
    )(pt, post)
    return jnp.transpose(out, (0, 2, 1))
